# Initial kernel scaffold; baseline (speedup 1.0000x reference)
import jax
import jax.numpy as jnp
from jax.experimental import pallas as pl


def kernel(x0, x1, x2, edge_index0, edge_index1, edge_index2, batch0, batch1, batch2, Wrel, brel, Wroot, pw, W0, b0, W1, b1, W2, b2):
    raise NotImplementedError("write your pallas kernel here")


# speedup vs baseline: 4.5537x; 4.5537x over previous
import functools

import jax
import jax.numpy as jnp
from jax.experimental import pallas as pl

_KS = [8000, 6400, 5120]


def _mlp_body(r_ref, W0_ref, b0_ref, W1_ref, b1_ref, W2_ref, b2_ref, o_ref):
    r = r_ref[...]
    h = b0_ref[...].reshape(1, -1)
    for b in range(3):
        h = h + jnp.dot(r[b:b + 1, :], W0_ref[b], preferred_element_type=jnp.float32)
    h = jax.nn.relu(h)
    h = jax.nn.relu(jnp.dot(h, W1_ref[...], preferred_element_type=jnp.float32) + b1_ref[...].reshape(1, -1))
    h = jnp.dot(h, W2_ref[...], preferred_element_type=jnp.float32) + b2_ref[...].reshape(1, -1)
    o_ref[...] = h / 1000.0


def _mlp(r, W0, b0, W1, b1, W2, b2):
    W0r = W0.reshape(3, 256, 512)
    return pl.pallas_call(
        _mlp_body,
        out_shape=jax.ShapeDtypeStruct((1, 10), jnp.float32),
    )(r, W0r, b0, W1, b1, W2, b2)


def _branch(x, ei, b, Wrel, brel, Wroot, pw):
    src = ei[0]
    dst = ei[1]
    n = x.shape[0]
    alive = jnp.ones((n,), dtype=jnp.bool_)
    outs = []
    for l in range(3):
        msg = x[src]
        agg = jax.ops.segment_sum(msg, dst, num_segments=n)
        h = jax.nn.relu(agg @ Wrel[b, l] + brel[b, l] + x @ Wroot[b, l])
        p = pw[b, l]
        score = (h @ p) / jnp.linalg.norm(p)
        ms = jnp.where(alive, score, -1e30)
        k = _KS[l]
        thresh = jax.lax.top_k(ms, k)[0][k - 1]
        kept = ms >= thresh
        x = jnp.where(kept[:, None], h * jnp.tanh(score)[:, None], 0.0)
        alive = kept
        mx = jnp.max(jnp.where(kept[:, None], x, -1e30), axis=0)
        mn = jnp.sum(x, axis=0) / k
        outs.append(jnp.concatenate([mx, mn])[None, :])
    return outs[0] + outs[1] + outs[2]


def kernel(x0, x1, x2, edge_index0, edge_index1, edge_index2, batch0, batch1,
           batch2, Wrel, brel, Wroot, pw, W0, b0, W1, b1, W2, b2):
    h0 = _branch(x0, edge_index0, 0, Wrel, brel, Wroot, pw)
    h1 = _branch(x1, edge_index1, 1, Wrel, brel, Wroot, pw)
    h2 = _branch(x2, edge_index2, 2, Wrel, brel, Wroot, pw)
    r = jnp.concatenate([h0, h1, h2], axis=0)
    return _mlp(r, W0, b0, W1, b1, W2, b2)


# speedup vs baseline: 18.5130x; 4.0655x over previous
import functools

import jax
import jax.numpy as jnp
from jax import lax
from jax.experimental import pallas as pl
from jax.experimental.pallas import tpu as pltpu
from jax.experimental.pallas import tpu_sc as plsc

_KS = [8000, 6400, 5120]
_N = 10000
_NP = 10240
_E = 320000
_D = 128
_NEG = -1e30


_EROWS = _E // 128
_QR = 2
_NQ = _EROWS // _QR
_NTILES = 32
_QFULL = _NQ // _NTILES
_QEXTRA = _NQ - _QFULL * _NTILES
_RPT = _NP // 16


def _segsum_body(x0, x1, x2, s0, d0, s1, d1, s2, d2, zeros, out,
                 sidx, sdst, rows_v, agg_sh, sem):
    c = lax.axis_index("c")
    s = lax.axis_index("s")
    w = s * 2 + c
    ng = jnp.where(w < _QEXTRA, _QFULL + 1, _QFULL)
    xs = (x0, x1, x2)
    srcs = (s0, s1, s2)
    dsts = (d0, d1, d2)
    for b in range(3):
        pltpu.sync_copy(zeros.at[pl.ds(s * _RPT, _RPT)],
                        agg_sh.at[pl.ds(s * _RPT, _RPT)])
        plsc.subcore_barrier()

        def group(j, _):
            qi = w + _NTILES * j
            pltpu.sync_copy(srcs[b].at[pl.ds(_QR * qi, _QR)], sidx)
            pltpu.sync_copy(dsts[b].at[pl.ds(_QR * qi, _QR)], sdst)
            gd = [pltpu.async_copy(xs[b].at[sidx.at[r]],
                                   rows_v.at[pl.ds(r * 128, 128)], sem)
                  for r in range(_QR)]
            for d in gd:
                d.wait()
            sd = [pltpu.async_copy(rows_v.at[pl.ds(r * 128, 128)],
                                   agg_sh.at[sdst.at[r]], sem, add=True)
                  for r in range(_QR)]
            for d in sd:
                d.wait()
            return 0

        lax.fori_loop(0, ng, group, 0)
        plsc.subcore_barrier()
        pltpu.sync_copy(
            agg_sh.at[pl.ds(s * _RPT, _RPT)],
            out.at[pl.ds((2 * b + c) * _NP + s * _RPT, _RPT)])
        plsc.subcore_barrier()


def _make_segsum():
    mesh = plsc.VectorSubcoreMesh(core_axis_name="c", subcore_axis_name="s")
    return pl.kernel(
        _segsum_body,
        out_type=jax.ShapeDtypeStruct((6 * _NP, _D), jnp.float32),
        mesh=mesh,
        scratch_types=[
            pltpu.VMEM((_QR, 128), jnp.int32),
            pltpu.VMEM((_QR, 128), jnp.int32),
            pltpu.VMEM((_QR * 128, _D), jnp.float32),
            pltpu.VMEM_SHARED((_NP, _D), jnp.float32),
            pltpu.SemaphoreType.DMA,
        ],
    )


def _sortable(bits):
    return jnp.where(bits >= 0, bits, bits ^ jnp.int32(0x7FFFFFFF))


def _layer_body(k, agg_ref, x_ref, ar_ref, Wrel_ref, brel_ref,
                Wroot_ref, pcol_ref, xo_ref, aro_ref, r_ref):
    agg = agg_ref[0] + agg_ref[1]
    x = x_ref[...]
    Wr = Wrel_ref[0, 0]
    Wo = Wroot_ref[0, 0]
    brel = brel_ref[0, 0]
    pcol = pcol_ref[0, 0]
    hp = lax.Precision.HIGHEST
    h = jnp.maximum(
        jnp.dot(agg, Wr, preferred_element_type=jnp.float32, precision=hp)
        + jnp.dot(x, Wo, preferred_element_type=jnp.float32, precision=hp)
        + brel, 0.0)
    inv = lax.rsqrt(jnp.sum(pcol * pcol))
    prow = pcol.reshape(1, _D)
    ht = h.T
    score_r = jnp.dot(prow, ht, preferred_element_type=jnp.float32,
                      precision=hp) * inv

    ar = ar_ref[...]
    keys_r = _sortable(lax.bitcast_convert_type(
        jnp.where(ar > 0, score_r, _NEG), jnp.int32))

    def cnt(t):
        return jnp.sum((keys_r >= t).astype(jnp.int32))

    cur0 = jnp.where(cnt(jnp.int32(0)) >= k, jnp.int32(0),
                     jnp.int32(-2147483648))

    def step(i, cur):
        trial = cur | (jnp.int32(1) << (30 - i))
        return jnp.where(cnt(trial) >= k, trial, cur)

    T = lax.fori_loop(0, 31, step, cur0)

    kept_r = keys_r >= T
    factor_r = jnp.tanh(score_r) * kept_r.astype(jnp.float32)
    xn_t = ht * factor_r
    xo_ref[...] = xn_t.T
    aro_ref[...] = kept_r.astype(jnp.bfloat16)
    mx = jnp.max(jnp.where(kept_r, xn_t, _NEG), axis=1)
    sm = jnp.sum(xn_t, axis=1) * (1.0 / k)
    r_ref[...] = jnp.concatenate([mx, sm])[None, :]


def _layer_call(b, l, agg_all, x, ar, Wrel, brel, Wroot, pwcol):
    f32 = jnp.float32
    return pl.pallas_call(
        functools.partial(_layer_body, _KS[l]),
        grid=(1,),
        in_specs=[
            pl.BlockSpec((2, _NP, _D), lambda i: (b, 0, 0)),
            pl.BlockSpec((_NP, _D), lambda i: (0, 0)),
            pl.BlockSpec((1, _NP), lambda i: (0, 0)),
            pl.BlockSpec((1, 1, _D, _D), lambda i: (b, l, 0, 0)),
            pl.BlockSpec((1, 1, 1, _D), lambda i: (b, l, 0, 0)),
            pl.BlockSpec((1, 1, _D, _D), lambda i: (b, l, 0, 0)),
            pl.BlockSpec((1, 1, _D, 1), lambda i: (b, l, 0, 0)),
        ],
        out_specs=[
            pl.BlockSpec((_NP, _D), lambda i: (0, 0)),
            pl.BlockSpec((1, _NP), lambda i: (0, 0)),
            pl.BlockSpec((1, 256), lambda i: (0, 0)),
        ],
        out_shape=[
            jax.ShapeDtypeStruct((_NP, _D), f32),
            jax.ShapeDtypeStruct((1, _NP), jnp.bfloat16),
            jax.ShapeDtypeStruct((1, 256), f32),
        ],
    )(agg_all, x, ar, Wrel, brel, Wroot, pwcol)


def _mlp_body(r_ref, W0_ref, b0_ref, W1_ref, b1_ref, W2_ref, b2_ref, o_ref):
    r = r_ref[...]
    hp = lax.Precision.HIGHEST
    h = b0_ref[...].reshape(1, -1)
    for b in range(3):
        h = h + jnp.dot(r[b:b + 1, :], W0_ref[b],
                        preferred_element_type=jnp.float32, precision=hp)
    h = jax.nn.relu(h)
    h = jax.nn.relu(jnp.dot(h, W1_ref[...], preferred_element_type=jnp.float32,
                            precision=hp) + b1_ref[...].reshape(1, -1))
    h = jnp.dot(h, W2_ref[...], preferred_element_type=jnp.float32,
                precision=hp) + b2_ref[...].reshape(1, -1)
    o_ref[...] = h / 1000.0


def _mlp(r, W0, b0, W1, b1, W2, b2):
    W0r = W0.reshape(3, 256, 512)
    return pl.pallas_call(
        _mlp_body,
        out_shape=jax.ShapeDtypeStruct((1, 10), jnp.float32),
    )(r, W0r, b0, W1, b1, W2, b2)


def kernel(x0, x1, x2, edge_index0, edge_index1, edge_index2, batch0, batch1,
           batch2, Wrel, brel, Wroot, pw, W0, b0, W1, b1, W2, b2):
    f32 = jnp.float32
    segsum = _make_segsum()
    pad = jnp.zeros((_NP - _N, _D), f32)
    xs = [jnp.concatenate([x, pad], axis=0) for x in (x0, x1, x2)]
    srcs = [ei[0].reshape(_EROWS, 128)
            for ei in (edge_index0, edge_index1, edge_index2)]
    dsts = [ei[1].reshape(_EROWS, 128)
            for ei in (edge_index0, edge_index1, edge_index2)]
    zeros = jnp.zeros((_NP, _D), f32)
    alive0 = jnp.concatenate(
        [jnp.ones((_N,), jnp.bfloat16), jnp.zeros((_NP - _N,), jnp.bfloat16)])
    ars = [alive0.reshape(1, _NP)] * 3
    pwcol = pw.reshape(3, 3, _D, 1)
    brel = brel.reshape(3, 3, 1, _D)

    r_acc = [None, None, None]
    for l in range(3):
        agg = segsum(xs[0], xs[1], xs[2], srcs[0], dsts[0], srcs[1], dsts[1],
                     srcs[2], dsts[2], zeros)
        agg = agg.reshape(6, _NP, _D)
        for b in range(3):
            xn, arn, r = _layer_call(b, l, agg, xs[b], ars[b],
                                     Wrel, brel, Wroot, pwcol)
            xs[b], ars[b] = xn, arn
            r_acc[b] = r if r_acc[b] is None else r_acc[b] + r
    rr = jnp.concatenate(r_acc, axis=0)
    return _mlp(rr, W0, b0, W1, b1, W2, b2)
